# Initial kernel scaffold; baseline (speedup 1.0000x reference)
import functools

import jax
import jax.numpy as jnp
from jax import lax
from jax.experimental import pallas as pl
from jax.experimental.pallas import tpu as pltpu
from jax.experimental.pallas import tpu_sc as plsc

N = 10000
E = 320000
D = 128

NC = 2
NS = 16
NW = NC * NS

NPAD = 10240
RPT = NPAD // NS
CH = 128
EPAD = 323584
EPW = EPAD // NW
NCHUNK = EPW // CH
RCH = RPT // CH
DEGW = 16

_mesh = plsc.VectorSubcoreMesh(core_axis_name="c", subcore_axis_name="s")


def _wid(cid, sid):
    return cid * NS + sid


@functools.partial(
    pl.kernel,
    out_type=[
        jax.ShapeDtypeStruct((NC, NPAD, DEGW), jnp.float32),
        jax.ShapeDtypeStruct((NC, NPAD, DEGW), jnp.float32),
    ],
    mesh=_mesh,
    scratch_types=[
        pltpu.VMEM((CH,), jnp.int32),
        pltpu.VMEM((CH,), jnp.int32),
        pltpu.VMEM((CH, DEGW), jnp.float32),
        pltpu.VMEM((RPT, DEGW), jnp.float32),
        pltpu.VMEM_SHARED((NPAD, DEGW), jnp.float32),
        pltpu.VMEM_SHARED((NPAD, DEGW), jnp.float32),
    ],
)
def _deg_kernel(src_hbm, dst_hbm, ones_hbm, zrow_hbm,
                dego_hbm, degi_hbm,
                sb, db, ob, rb, dego_sh, degi_sh):
    cid = lax.axis_index("c")
    sid = lax.axis_index("s")
    wid = _wid(cid, sid)
    r0 = sid * RPT

    pltpu.sync_copy(zrow_hbm, rb)
    pltpu.sync_copy(rb, dego_sh.at[pl.ds(r0, RPT)])
    pltpu.sync_copy(rb, degi_sh.at[pl.ds(r0, RPT)])
    pltpu.sync_copy(ones_hbm, ob)
    plsc.subcore_barrier()

    ebase = wid * EPW

    def body(g, carry):
        base = ebase + g * CH
        pltpu.sync_copy(src_hbm.at[pl.ds(base, CH)], sb)
        pltpu.sync_copy(dst_hbm.at[pl.ds(base, CH)], db)
        pltpu.sync_copy(ob, dego_sh.at[sb], add=True)
        pltpu.sync_copy(ob, degi_sh.at[db], add=True)
        return carry

    lax.fori_loop(0, NCHUNK, body, 0)
    plsc.subcore_barrier()

    pltpu.sync_copy(dego_sh.at[pl.ds(r0, RPT)], rb)
    pltpu.sync_copy(rb, dego_hbm.at[cid, pl.ds(r0, RPT)])
    pltpu.sync_copy(degi_sh.at[pl.ds(r0, RPT)], rb)
    pltpu.sync_copy(rb, degi_hbm.at[cid, pl.ds(r0, RPT)])


@functools.partial(
    pl.kernel,
    out_type=jax.ShapeDtypeStruct((NC, NPAD, D), jnp.float32),
    mesh=_mesh,
    scratch_types=[
        pltpu.VMEM((CH,), jnp.int32),
        pltpu.VMEM((CH,), jnp.int32),
        pltpu.VMEM((CH, D), jnp.float32),
        pltpu.VMEM_SHARED((NPAD, D), jnp.float32),
        pltpu.SemaphoreType.DMA,
    ],
)
def _agg_kernel(src_hbm, dst_hbm, x_hbm, zc_hbm, out_hbm,
                sb, db, rb, agg_sh, sem):
    cid = lax.axis_index("c")
    sid = lax.axis_index("s")
    wid = _wid(cid, sid)
    r0 = sid * RPT

    pltpu.sync_copy(zc_hbm, rb)
    for j in range(RCH):
        pltpu.sync_copy(rb, agg_sh.at[pl.ds(r0 + j * CH, CH)])
    plsc.subcore_barrier()

    ebase = wid * EPW

    def body(g, carry):
        base = ebase + g * CH
        pltpu.sync_copy(src_hbm.at[pl.ds(base, CH)], sb)
        pltpu.sync_copy(dst_hbm.at[pl.ds(base, CH)], db)
        pltpu.async_copy(x_hbm.at[sb], rb, sem).wait()
        pltpu.sync_copy(rb, agg_sh.at[db], add=True)
        return carry

    lax.fori_loop(0, NCHUNK, body, 0)
    plsc.subcore_barrier()

    for j in range(RCH):
        pltpu.sync_copy(agg_sh.at[pl.ds(r0 + j * CH, CH)], rb)
        pltpu.sync_copy(rb, out_hbm.at[cid, pl.ds(r0 + j * CH, CH)])


def _rsqrt_deg(d0, d1):
    deg = jnp.sum(d0[0] + d1[0], axis=-1, keepdims=True)
    return lax.rsqrt(jnp.maximum(deg, 1.0))


def _scale_body(h_ref, dego0_ref, dego1_ref, o_ref):
    do = _rsqrt_deg(dego0_ref[...], dego1_ref[...])
    o_ref[...] = h_ref[...] * do


def _layer_body(relu, scale_out, agg0_ref, agg1_ref, degi0_ref, degi1_ref,
                dego0_ref, dego1_ref, w_ref, b_ref, o_ref):
    di = _rsqrt_deg(degi0_ref[...], degi1_ref[...])
    x = (agg0_ref[0] + agg1_ref[0]) * di
    y = jnp.dot(x, w_ref[...], preferred_element_type=jnp.float32) + b_ref[...]
    if relu:
        y = jnp.maximum(y, 0.0)
    if scale_out:
        do = _rsqrt_deg(dego0_ref[...], dego1_ref[...])
        rows = pl.program_id(0) * RPT + lax.broadcasted_iota(
            jnp.int32, (RPT, 1), 0)
        y = jnp.where(rows < N, y * do, 0.0)
    o_ref[...] = y


def _row_spec(width):
    return pl.BlockSpec((RPT, width), lambda i: (i, 0))


def _part_spec(width):
    return pl.BlockSpec((1, RPT, width), lambda i: (0, i, 0))


def _part2_spec(width):
    return pl.BlockSpec((1, RPT, width), lambda i: (1, i, 0))


def _scale_x(h_pad, dego):
    return pl.pallas_call(
        _scale_body,
        grid=(NPAD // RPT,),
        in_specs=[_row_spec(D), _part_spec(DEGW), _part2_spec(DEGW)],
        out_specs=_row_spec(D),
        out_shape=jax.ShapeDtypeStruct((NPAD, D), jnp.float32),
    )(h_pad, dego, dego)


def _layer(agg, degi, dego, w, b, relu, scale_out):
    body = functools.partial(_layer_body, relu, scale_out)
    return pl.pallas_call(
        body,
        grid=(NPAD // RPT,),
        in_specs=[
            _part_spec(D), _part2_spec(D),
            _part_spec(DEGW), _part2_spec(DEGW),
            _part_spec(DEGW), _part2_spec(DEGW),
            pl.BlockSpec((D, D), lambda i: (0, 0)),
            pl.BlockSpec((1, D), lambda i: (0, 0)),
        ],
        out_specs=_row_spec(D),
        out_shape=jax.ShapeDtypeStruct((NPAD, D), jnp.float32),
    )(agg, agg, degi, degi, dego, dego, w, b)


def kernel(h, edge_index, W1, b1, W2, b2):
    src = edge_index[0]
    dst = edge_index[1]
    pad_idx = N + (jnp.arange(EPAD - E, dtype=jnp.int32) % (NPAD - N))
    srcp = jnp.concatenate([src, pad_idx])
    dstp = jnp.concatenate([dst, pad_idx])
    h_pad = jnp.pad(h, ((0, NPAD - N), (0, 0)))

    ones_w = jnp.ones((CH, DEGW), jnp.float32)
    zrow = jnp.zeros((RPT, DEGW), jnp.float32)
    zc = jnp.zeros((CH, D), jnp.float32)
    b1r = b1.reshape(1, D)
    b2r = b2.reshape(1, D)

    dego, degi = _deg_kernel(srcp, dstp, ones_w, zrow)

    x1 = _scale_x(h_pad, dego)
    agg1 = _agg_kernel(srcp, dstp, x1, zc)
    x2 = _layer(agg1, degi, dego, W1, b1r, relu=True, scale_out=True)
    agg2 = _agg_kernel(srcp, dstp, x2, zc)
    out = _layer(agg2, degi, dego, W2, b2r, relu=False, scale_out=False)
    return out[:N]


# speedup vs baseline: 4.9995x; 4.9995x over previous
import functools

import jax
import jax.numpy as jnp
from jax import lax
from jax.experimental import pallas as pl
from jax.experimental.pallas import tpu as pltpu
from jax.experimental.pallas import tpu_sc as plsc

N = 10000
E = 320000
D = 128

NC = 2
NS = 16
NW = NC * NS

NPAD = 10240
RPT = NPAD // NS
CH = 128
EPAD = 323584
EPW = EPAD // NW
NCHUNK = EPW // CH
RCH = RPT // CH

DEGW = 16

_mesh = plsc.VectorSubcoreMesh(core_axis_name="c", subcore_axis_name="s")
_linear = pltpu.CompilerParams(use_tc_tiling_on_sc=False)


@functools.partial(
    pl.kernel,
    out_type=[
        jax.ShapeDtypeStruct((NC, NPAD, DEGW), jnp.float32),
        jax.ShapeDtypeStruct((NC, NPAD, DEGW), jnp.float32),
    ],
    mesh=_mesh,
    compiler_params=_linear,
    scratch_types=[
        pltpu.VMEM((1, CH), jnp.int32),
        pltpu.VMEM((1, CH), jnp.int32),
        pltpu.VMEM((CH, DEGW), jnp.float32),
        pltpu.VMEM((RPT, DEGW), jnp.float32),
        pltpu.VMEM_SHARED((NPAD, DEGW), jnp.float32),
        pltpu.VMEM_SHARED((NPAD, DEGW), jnp.float32),
    ],
)
def _deg_kernel(src_hbm, dst_hbm, ones_hbm, zrow_hbm,
                dego_hbm, degi_hbm,
                sb, db, ob, rb, dego_sh, degi_sh):
    cid = lax.axis_index("c")
    sid = lax.axis_index("s")
    wid = cid * NS + sid
    r0 = sid * RPT

    pltpu.sync_copy(zrow_hbm, rb)
    pltpu.sync_copy(rb, dego_sh.at[pl.ds(r0, RPT)])
    pltpu.sync_copy(rb, degi_sh.at[pl.ds(r0, RPT)])
    pltpu.sync_copy(ones_hbm, ob)
    plsc.subcore_barrier()

    ebase = wid * EPW

    def body(g, carry):
        base = ebase + g * CH
        pltpu.sync_copy(src_hbm.at[pl.ds(base, CH)], sb.at[0])
        pltpu.sync_copy(dst_hbm.at[pl.ds(base, CH)], db.at[0])
        pltpu.sync_copy(ob, dego_sh.at[sb.at[0]], add=True)
        pltpu.sync_copy(ob, degi_sh.at[db.at[0]], add=True)
        return carry

    lax.fori_loop(0, NCHUNK, body, 0)
    plsc.subcore_barrier()

    pltpu.sync_copy(dego_sh.at[pl.ds(r0, RPT)], rb)
    pltpu.sync_copy(rb, dego_hbm.at[cid, pl.ds(r0, RPT)])
    pltpu.sync_copy(degi_sh.at[pl.ds(r0, RPT)], rb)
    pltpu.sync_copy(rb, degi_hbm.at[cid, pl.ds(r0, RPT)])


@functools.partial(
    pl.kernel,
    out_type=jax.ShapeDtypeStruct((NC, NPAD, D), jnp.float32),
    mesh=_mesh,
    scratch_types=[
        pltpu.VMEM((1, CH), jnp.int32),
        pltpu.VMEM((1, CH), jnp.int32),
        pltpu.VMEM((CH, D), jnp.float32),
        pltpu.VMEM_SHARED((NPAD, D), jnp.float32),
        pltpu.SemaphoreType.DMA,
    ],
)
def _agg_kernel(src_hbm, dst_hbm, x_hbm, zc_hbm, out_hbm,
                sb, db, rb, agg_sh, sem):
    cid = lax.axis_index("c")
    sid = lax.axis_index("s")
    wid = cid * NS + sid
    r0 = sid * RPT

    pltpu.sync_copy(zc_hbm, rb)
    for j in range(RCH):
        pltpu.sync_copy(rb, agg_sh.at[pl.ds(r0 + j * CH, CH)])
    plsc.subcore_barrier()

    ebase = wid * EPW

    def body(g, carry):
        base = ebase + g * CH
        pltpu.sync_copy(src_hbm.at[pl.ds(base, CH)], sb.at[0])
        pltpu.sync_copy(dst_hbm.at[pl.ds(base, CH)], db.at[0])
        pltpu.async_copy(x_hbm.at[sb.at[0]], rb, sem).wait()
        pltpu.sync_copy(rb, agg_sh.at[db.at[0]], add=True)
        return carry

    lax.fori_loop(0, NCHUNK, body, 0)
    plsc.subcore_barrier()

    for j in range(RCH):
        pltpu.sync_copy(agg_sh.at[pl.ds(r0 + j * CH, CH)], rb)
        pltpu.sync_copy(rb, out_hbm.at[cid, pl.ds(r0 + j * CH, CH)])


def _rsqrt_deg(d0, d1):
    deg = d0[0, :, 0:1] + d1[0, :, 0:1]
    return lax.rsqrt(jnp.maximum(deg, 1.0))


def _scale_body(h_ref, dego0_ref, dego1_ref, o_ref):
    do = _rsqrt_deg(dego0_ref[...], dego1_ref[...])
    o_ref[...] = h_ref[...] * do


def _layer_body(relu, scale_out, agg0_ref, agg1_ref, degi0_ref, degi1_ref,
                dego0_ref, dego1_ref, w_ref, b_ref, o_ref):
    di = _rsqrt_deg(degi0_ref[...], degi1_ref[...])
    x = (agg0_ref[0] + agg1_ref[0]) * di
    y = jnp.dot(x, w_ref[...], preferred_element_type=jnp.float32) + b_ref[...]
    if relu:
        y = jnp.maximum(y, 0.0)
    if scale_out:
        do = _rsqrt_deg(dego0_ref[...], dego1_ref[...])
        rows = pl.program_id(0) * RPT + lax.broadcasted_iota(
            jnp.int32, (RPT, 1), 0)
        y = jnp.where(rows < N, y * do, 0.0)
    o_ref[...] = y


def _row_spec(width):
    return pl.BlockSpec((RPT, width), lambda i: (i, 0))


def _part_spec(width):
    return pl.BlockSpec((1, RPT, width), lambda i: (0, i, 0))


def _part2_spec(width):
    return pl.BlockSpec((1, RPT, width), lambda i: (1, i, 0))


def _scale_x(h_pad, dego):
    return pl.pallas_call(
        _scale_body,
        grid=(NPAD // RPT,),
        in_specs=[_row_spec(D), _part_spec(DEGW), _part2_spec(DEGW)],
        out_specs=_row_spec(D),
        out_shape=jax.ShapeDtypeStruct((NPAD, D), jnp.float32),
    )(h_pad, dego, dego)


def _layer(agg, degi, dego, w, b, relu, scale_out):
    body = functools.partial(_layer_body, relu, scale_out)
    return pl.pallas_call(
        body,
        grid=(NPAD // RPT,),
        in_specs=[
            _part_spec(D), _part2_spec(D),
            _part_spec(DEGW), _part2_spec(DEGW),
            _part_spec(DEGW), _part2_spec(DEGW),
            pl.BlockSpec((D, D), lambda i: (0, 0)),
            pl.BlockSpec((1, D), lambda i: (0, 0)),
        ],
        out_specs=_row_spec(D),
        out_shape=jax.ShapeDtypeStruct((NPAD, D), jnp.float32),
    )(agg, agg, degi, degi, dego, dego, w, b)


def kernel(h, edge_index, W1, b1, W2, b2):
    src = edge_index[0]
    dst = edge_index[1]
    pad_idx = N + (jnp.arange(EPAD - E, dtype=jnp.int32) % (NPAD - N))
    srcp = jnp.concatenate([src, pad_idx])
    dstp = jnp.concatenate([dst, pad_idx])
    h_pad = jnp.pad(h, ((0, NPAD - N), (0, 0)))

    zc = jnp.zeros((CH, D), jnp.float32)
    ones_w = jnp.ones((CH, DEGW), jnp.float32)
    zrow = jnp.zeros((RPT, DEGW), jnp.float32)
    b1r = b1.reshape(1, D)
    b2r = b2.reshape(1, D)

    dego, degi = _deg_kernel(srcp, dstp, ones_w, zrow)

    x1 = _scale_x(h_pad, dego)
    agg1 = _agg_kernel(srcp, dstp, x1, zc)
    x2 = _layer(agg1, degi, dego, W1, b1r, relu=True, scale_out=True)
    agg2 = _agg_kernel(srcp, dstp, x2, zc)
    out = _layer(agg2, degi, dego, W2, b2r, relu=False, scale_out=False)
    return out[:N]
